# baseline (speedup 1.0000x reference)
import functools

import numpy as np

import jax
import jax.numpy as jnp
from jax import lax
from jax.experimental import pallas as pl
from jax.experimental.pallas import tpu as pltpu
from jax.experimental.pallas import tpu_sc as plsc

N_NODES = 10000
D = 128
N_PAD = 10240
E = 320000
E_PAD = 327680
NC, NS = 2, 16
NW = NC * NS
EPT = E_PAD // NW
CHUNK = 128
NCHUNK = EPT // CHUNK
RPT = N_PAD // NS
BLK = 2048
GRID = N_PAD // BLK

_PC = np.empty((D,), np.int32)
for _g in range(D // 32):
    for _j in range(16):
        for _e in range(2):
            _PC[32 * _g + 2 * _j + _e] = 32 * _g + 16 * _e + _j
_PMT = np.zeros((D, D), np.float32)
_PMT[np.arange(D), _PC] = 1.0

_mesh = plsc.VectorSubcoreMesh(core_axis_name="c", subcore_axis_name="s")


EPT31 = E - 31 * EPT


@functools.partial(
    pl.kernel,
    out_type=(jax.ShapeDtypeStruct((8, N_PAD), jnp.float32),
              jax.ShapeDtypeStruct((NS, 2, NCHUNK, CHUNK), jnp.int32)),
    mesh=_mesh,
    scratch_types=[
        pltpu.VMEM((EPT,), jnp.int32),
        pltpu.VMEM((EPT,), jnp.int32),
        pltpu.VMEM((NCHUNK, CHUNK), jnp.int32),
        pltpu.VMEM((NCHUNK, CHUNK), jnp.int32),
        pltpu.VMEM((CHUNK,), jnp.float32),
        pltpu.VMEM_SHARED((N_PAD,), jnp.float32),
        pltpu.SemaphoreType.DMA,
        pltpu.SemaphoreType.DMA,
    ],
)
def _sc_prep(ei_hbm, zeros_hbm, out_hbm, packed_out, src_all, dst_all,
             dst2, packed_all, onesv, shared_deg, sa, sb):
    cid = lax.axis_index("c")
    sid = lax.axis_index("s")
    wid = cid * NS + sid
    base = wid * EPT

    @pl.when(wid < NW - 1)
    def _():
        pltpu.sync_copy(ei_hbm.at[0, pl.ds(base, EPT)], src_all)
        pltpu.sync_copy(ei_hbm.at[1, pl.ds(base, EPT)], dst_all)

    @pl.when(wid == NW - 1)
    def _():
        pltpu.sync_copy(ei_hbm.at[0, pl.ds(base, EPT31)],
                        src_all.at[pl.ds(0, EPT31)])
        pltpu.sync_copy(ei_hbm.at[1, pl.ds(base, EPT31)],
                        dst_all.at[pl.ds(0, EPT31)])

        def fill(i, carry):
            src_all[pl.ds(EPT31 + i * 16, 16)] = jnp.zeros((16,), jnp.int32)
            dst_all[pl.ds(EPT31 + i * 16, 16)] = jnp.full(
                (16,), N_PAD - 1, jnp.int32)
            return carry

        lax.fori_loop(0, (EPT - EPT31) // 16, fill, 0)

    pltpu.sync_copy(zeros_hbm.at[pl.ds(sid * RPT, RPT)],
                    shared_deg.at[pl.ds(sid * RPT, RPT)])
    for k in range(CHUNK // 16):
        onesv[pl.ds(k * 16, 16)] = jnp.ones((16,), jnp.float32)

    def pack(g, carry):
        for k in range(CHUNK // 16):
            sv = src_all[pl.ds(g * CHUNK + k * 16, 16)]
            dv = dst_all[pl.ds(g * CHUNK + k * 16, 16)]
            dst2[g, pl.ds(k * 16, 16)] = dv
            packed_all[g, pl.ds(k * 16, 16)] = sv | lax.shift_left(dv, 14)
        return carry

    lax.fori_loop(0, NCHUNK, pack, 0)
    pltpu.sync_copy(packed_all, packed_out.at[wid // 2, wid % 2])
    plsc.subcore_barrier()

    sems = (sa, sb)
    NGRP = NCHUNK // 8

    def fire(t):
        for j in range(8):
            pltpu.async_copy(onesv, shared_deg.at[dst2.at[t * 8 + j]],
                             sems[t % 2], add=True)

    def drain(t):
        for j in range(8):
            pltpu.make_async_copy(
                onesv, shared_deg.at[dst2.at[t * 8 + j]],
                sems[t % 2]).wait()

    fire(0)
    for t in range(1, NGRP):
        fire(t)
        drain(t - 1)
    drain(NGRP - 1)

    plsc.subcore_barrier()
    pltpu.sync_copy(shared_deg.at[pl.ds(sid * RPT, RPT)],
                    out_hbm.at[cid, pl.ds(sid * RPT, RPT)])


DH = D // 2
NBUF = 4
EPS = E_PAD // NS
NPHASE = 2


@functools.partial(
    pl.kernel,
    out_type=jax.ShapeDtypeStruct((N_PAD, D), jnp.float32),
    mesh=_mesh,
    scratch_types=[
        pltpu.VMEM((NCHUNK, CHUNK), jnp.int32),
        [pltpu.VMEM((CHUNK,), jnp.int32) for _ in range(NBUF)],
        [pltpu.VMEM((CHUNK,), jnp.int32) for _ in range(NBUF)],
        [pltpu.VMEM((CHUNK, DH), jnp.bfloat16) for _ in range(NBUF)],
        [pltpu.VMEM((CHUNK, DH), jnp.float32) for _ in range(NBUF)],
        pltpu.VMEM_SHARED((N_PAD, DH), jnp.bfloat16),
        pltpu.VMEM_SHARED((N_PAD, DH), jnp.float32),
        [pltpu.SemaphoreType.DMA for _ in range(NBUF)],
        [pltpu.SemaphoreType.DMA for _ in range(NBUF)],
        pltpu.SemaphoreType.DMA,
        pltpu.SemaphoreType.DMA,
    ],
    compiler_params=pltpu.CompilerParams(use_tc_tiling_on_sc=False,
                                        needs_layout_passes=False),
)
def _sc_scatter(packed_hbm, h_hbm, zrows_hbm, out_hbm,
                packed_all, srcv, dstv, rows, rowsf, h_sh, acc,
                sg, ss, st0, st1):
    cid = lax.axis_index("c")
    sid = lax.axis_index("s")
    pltpu.async_copy(
        h_hbm.at[pl.ds(sid * RPT, RPT), pl.ds(cid * DH, DH)],
        h_sh.at[pl.ds(sid * RPT, RPT)], st0)
    pltpu.async_copy(zrows_hbm, acc.at[pl.ds(sid * RPT, RPT)], st1)
    pltpu.sync_copy(packed_hbm.at[sid, 0], packed_all)
    pltpu.make_async_copy(
        h_hbm.at[pl.ds(sid * RPT, RPT), pl.ds(cid * DH, DH)],
        h_sh.at[pl.ds(sid * RPT, RPT)], st0).wait()
    pltpu.make_async_copy(zrows_hbm, acc.at[pl.ds(sid * RPT, RPT)],
                          st1).wait()
    plsc.subcore_barrier()

    def unpack(g, dst_ref, shift):
        for k in range(CHUNK // 16):
            w = packed_all[g, pl.ds(k * 16, 16)]
            dst_ref[pl.ds(k * 16, 16)] = (
                lax.shift_right_logical(w, shift) & 16383)

    for phase in range(NPHASE):
        if phase:
            pltpu.sync_copy(packed_hbm.at[sid, phase], packed_all)
        for b in range(2):
            unpack(b, srcv[b], 0)
            pltpu.async_copy(h_sh.at[srcv[b]], rows[b], sg[b])

        def conv(b):
            def crow(r4, carry):
                for u in range(4):
                    r = r4 * 4 + u
                    for c in range(DH // 32):
                        w = plsc.bitcast(rows[b][r, pl.ds(c * 32, 32)],
                                         jnp.int32)
                        rowsf[b][r, pl.ds(c * 32, 16)] = plsc.bitcast(
                            w << 16, jnp.float32)
                        rowsf[b][r, pl.ds(c * 32 + 16, 16)] = plsc.bitcast(
                            w & jnp.int32(-65536), jnp.float32)
                return carry

            lax.fori_loop(0, CHUNK // 4, crow, 0)

        def body(t, carry):
            for j in range(NBUF):
                p = t * NBUF + j
                b = j
                c = (j + 2) % NBUF
                pltpu.make_async_copy(h_sh.at[srcv[b]], rows[b],
                                      sg[b]).wait()
                unpack(p, dstv[b], 14)
                conv(b)
                pltpu.async_copy(rowsf[b], acc.at[dstv[b]], ss[b], add=True)

                @pl.when(p >= 2)
                def _():
                    pltpu.make_async_copy(rowsf[c], acc.at[dstv[c]],
                                          ss[c]).wait()

                @pl.when(p + 2 < NCHUNK)
                def _():
                    unpack(p + 2, srcv[c], 0)
                    pltpu.async_copy(h_sh.at[srcv[c]], rows[c], sg[c])
            return carry

        lax.fori_loop(0, NCHUNK // NBUF, body, 0)
        for p in (NCHUNK - 2, NCHUNK - 1):
            b = p % NBUF
            pltpu.make_async_copy(rowsf[b], acc.at[dstv[b]], ss[b]).wait()

    plsc.subcore_barrier()
    pltpu.sync_copy(
        acc.at[pl.ds(sid * RPT, RPT)],
        out_hbm.at[pl.ds(sid * RPT, RPT), pl.ds(cid * DH, DH)])


def _dis(degp_ref):
    return lax.rsqrt(degp_ref[0, :] + degp_ref[1, :] + 1.0)


def _tc_first(x_ref, w_ref, degp_ref, o_ref):
    h = jnp.dot(x_ref[...], w_ref[...], preferred_element_type=jnp.float32)
    o_ref[...] = (h * _dis(degp_ref)[:, None]).astype(jnp.bfloat16)


def _tc_mid(p_ref, h_ref, degp_ref, pmt_ref, w_ref, b_ref, o_ref):
    dis = _dis(degp_ref)
    h1o = jnp.dot(h_ref[...].astype(jnp.float32), pmt_ref[...],
                  preferred_element_type=jnp.float32)
    s = p_ref[...] + h1o
    a = jnp.maximum(s * dis[:, None] + b_ref[...], 0.0)
    o_ref[...] = (jnp.dot(a, w_ref[...], preferred_element_type=jnp.float32)
                  * dis[:, None]).astype(jnp.bfloat16)


def _tc_last(p_ref, h_ref, degp_ref, pmt_ref, b_ref, o_ref):
    h2o = jnp.dot(h_ref[...].astype(jnp.float32), pmt_ref[...],
                  preferred_element_type=jnp.float32)
    s = p_ref[...] + h2o
    o_ref[...] = s * _dis(degp_ref)[:, None] + b_ref[...]


_SPEC_ROWS = pl.BlockSpec((BLK, D), lambda i: (i, 0))
_SPEC_W = pl.BlockSpec((D, D), lambda i: (0, 0))
_SPEC_DEG = pl.BlockSpec((8, BLK), lambda i: (0, i))
_SPEC_B = pl.BlockSpec((1, D), lambda i: (0, 0))
_OUT_ROWS = jax.ShapeDtypeStruct((N_PAD, D), jnp.float32)
_OUT_BF = jax.ShapeDtypeStruct((N_PAD, D), jnp.bfloat16)


@jax.jit
def kernel(x, edge_index, W1, b1, W2, b2):
    ei = edge_index.astype(jnp.int32)
    x_pad = jnp.pad(x, ((0, N_PAD - N_NODES), (0, 0)))
    z1 = jnp.zeros((N_PAD,), jnp.float32)
    zrows = jnp.zeros((RPT, DH), jnp.float32)
    b1r = b1.reshape(1, D)
    b2r = b2.reshape(1, D)
    pc = jnp.asarray(_PC)
    pmt = jnp.asarray(_PMT)
    W1P = W1[:, pc]
    W2P = W2[:, pc]

    degp, packed3 = _sc_prep(ei, z1)

    h1 = pl.pallas_call(
        _tc_first,
        grid=(GRID,),
        in_specs=[_SPEC_ROWS, _SPEC_W, _SPEC_DEG],
        out_specs=_SPEC_ROWS,
        out_shape=_OUT_BF,
    )(x_pad, W1P, degp)

    p1 = _sc_scatter(packed3, h1, zrows)

    h2 = pl.pallas_call(
        _tc_mid,
        grid=(GRID,),
        in_specs=[_SPEC_ROWS, _SPEC_ROWS, _SPEC_DEG, _SPEC_W, _SPEC_W,
                  _SPEC_B],
        out_specs=_SPEC_ROWS,
        out_shape=_OUT_BF,
    )(p1, h1, degp, pmt, W2P, b1r)

    p2 = _sc_scatter(packed3, h2, zrows)

    out = pl.pallas_call(
        _tc_last,
        grid=(GRID,),
        in_specs=[_SPEC_ROWS, _SPEC_ROWS, _SPEC_DEG, _SPEC_W, _SPEC_B],
        out_specs=_SPEC_ROWS,
        out_shape=jax.ShapeDtypeStruct((N_NODES, D), jnp.float32),
    )(p2, h2, degp, pmt, b2r)

    return out


# speedup vs baseline: 2.1495x; 2.1495x over previous
import functools

import jax
import jax.numpy as jnp
from jax import lax
from jax.experimental import pallas as pl
from jax.experimental.pallas import tpu as pltpu
from jax.experimental.pallas import tpu_sc as plsc

N_NODES = 10000
D = 128
N_PAD = 10240
E = 320000
E_PAD = 327680
NC, NS = 2, 16
NW = NC * NS
EPT = E_PAD // NW
CHUNK = 128
NCHUNK = EPT // CHUNK
RPT = N_PAD // NS
BLK = 2048
GRID = N_PAD // BLK

_mesh = plsc.VectorSubcoreMesh(core_axis_name="c", subcore_axis_name="s")


EPT31 = E - 31 * EPT


@functools.partial(
    pl.kernel,
    out_type=(jax.ShapeDtypeStruct((8, N_PAD), jnp.float32),
              jax.ShapeDtypeStruct((NS, 2, NCHUNK, CHUNK), jnp.int32)),
    mesh=_mesh,
    scratch_types=[
        pltpu.VMEM((EPT,), jnp.int32),
        pltpu.VMEM((EPT,), jnp.int32),
        pltpu.VMEM((NCHUNK, CHUNK), jnp.int32),
        pltpu.VMEM((NCHUNK, CHUNK), jnp.int32),
        pltpu.VMEM((CHUNK,), jnp.float32),
        pltpu.VMEM_SHARED((N_PAD,), jnp.float32),
        pltpu.SemaphoreType.DMA,
        pltpu.SemaphoreType.DMA,
    ],
)
def _sc_prep(ei_hbm, zeros_hbm, out_hbm, packed_out, src_all, dst_all,
             dst2, packed_all, onesv, shared_deg, sa, sb):
    cid = lax.axis_index("c")
    sid = lax.axis_index("s")
    wid = cid * NS + sid
    base = wid * EPT

    @pl.when(wid < NW - 1)
    def _():
        pltpu.sync_copy(ei_hbm.at[0, pl.ds(base, EPT)], src_all)
        pltpu.sync_copy(ei_hbm.at[1, pl.ds(base, EPT)], dst_all)

    @pl.when(wid == NW - 1)
    def _():
        pltpu.sync_copy(ei_hbm.at[0, pl.ds(base, EPT31)],
                        src_all.at[pl.ds(0, EPT31)])
        pltpu.sync_copy(ei_hbm.at[1, pl.ds(base, EPT31)],
                        dst_all.at[pl.ds(0, EPT31)])

        def fill(i, carry):
            src_all[pl.ds(EPT31 + i * 16, 16)] = jnp.zeros((16,), jnp.int32)
            dst_all[pl.ds(EPT31 + i * 16, 16)] = jnp.full(
                (16,), N_PAD - 1, jnp.int32)
            return carry

        lax.fori_loop(0, (EPT - EPT31) // 16, fill, 0)

    pltpu.sync_copy(zeros_hbm.at[pl.ds(sid * RPT, RPT)],
                    shared_deg.at[pl.ds(sid * RPT, RPT)])
    for k in range(CHUNK // 16):
        onesv[pl.ds(k * 16, 16)] = jnp.ones((16,), jnp.float32)

    def pack(g, carry):
        for k in range(CHUNK // 16):
            sv = src_all[pl.ds(g * CHUNK + k * 16, 16)]
            dv = dst_all[pl.ds(g * CHUNK + k * 16, 16)]
            dst2[g, pl.ds(k * 16, 16)] = dv
            packed_all[g, pl.ds(k * 16, 16)] = sv | lax.shift_left(dv, 14)
        return carry

    lax.fori_loop(0, NCHUNK, pack, 0)
    pltpu.sync_copy(packed_all, packed_out.at[wid // 2, wid % 2])
    plsc.subcore_barrier()

    sems = (sa, sb)
    NGRP = NCHUNK // 8

    def fire(t):
        for j in range(8):
            pltpu.async_copy(onesv, shared_deg.at[dst2.at[t * 8 + j]],
                             sems[t % 2], add=True)

    def drain(t):
        for j in range(8):
            pltpu.make_async_copy(
                onesv, shared_deg.at[dst2.at[t * 8 + j]],
                sems[t % 2]).wait()

    fire(0)
    for t in range(1, NGRP):
        fire(t)
        drain(t - 1)
    drain(NGRP - 1)

    plsc.subcore_barrier()
    pltpu.sync_copy(shared_deg.at[pl.ds(sid * RPT, RPT)],
                    out_hbm.at[cid, pl.ds(sid * RPT, RPT)])


DH = D // 2
NBUF = 4
EPS = E_PAD // NS
NPHASE = 2


@functools.partial(
    pl.kernel,
    out_type=jax.ShapeDtypeStruct((N_PAD, D), jnp.bfloat16),
    mesh=_mesh,
    scratch_types=[
        pltpu.VMEM((NCHUNK, CHUNK), jnp.int32),
        [pltpu.VMEM((CHUNK,), jnp.int32) for _ in range(NBUF)],
        [pltpu.VMEM((CHUNK,), jnp.int32) for _ in range(NBUF)],
        [pltpu.VMEM((CHUNK, DH), jnp.bfloat16) for _ in range(NBUF)],
        pltpu.VMEM_SHARED((N_PAD, DH), jnp.bfloat16),
        pltpu.VMEM_SHARED((N_PAD, DH), jnp.bfloat16),
        [pltpu.SemaphoreType.DMA for _ in range(NBUF)],
        [pltpu.SemaphoreType.DMA for _ in range(NBUF)],
        pltpu.SemaphoreType.DMA,
        pltpu.SemaphoreType.DMA,
    ],
    compiler_params=pltpu.CompilerParams(use_tc_tiling_on_sc=False),
)
def _sc_scatter(packed_hbm, h_hbm, zrows_hbm, out_hbm,
                packed_all, srcv, dstv, rows, h_sh, acc, sg, ss, st0, st1):
    cid = lax.axis_index("c")
    sid = lax.axis_index("s")
    pltpu.async_copy(
        h_hbm.at[pl.ds(sid * RPT, RPT), pl.ds(cid * DH, DH)],
        h_sh.at[pl.ds(sid * RPT, RPT)], st0)
    pltpu.async_copy(zrows_hbm, acc.at[pl.ds(sid * RPT, RPT)], st1)
    pltpu.sync_copy(packed_hbm.at[sid, 0], packed_all)
    pltpu.make_async_copy(
        h_hbm.at[pl.ds(sid * RPT, RPT), pl.ds(cid * DH, DH)],
        h_sh.at[pl.ds(sid * RPT, RPT)], st0).wait()
    pltpu.make_async_copy(zrows_hbm, acc.at[pl.ds(sid * RPT, RPT)],
                          st1).wait()
    plsc.subcore_barrier()

    def unpack(g, dst_ref, shift):
        for k in range(CHUNK // 16):
            w = packed_all[g, pl.ds(k * 16, 16)]
            dst_ref[pl.ds(k * 16, 16)] = (
                lax.shift_right_logical(w, shift) & 16383)

    for phase in range(NPHASE):
        if phase:
            pltpu.sync_copy(packed_hbm.at[sid, phase], packed_all)
        for b in range(2):
            unpack(b, srcv[b], 0)
            pltpu.async_copy(h_sh.at[srcv[b]], rows[b], sg[b])

        def body(t, carry):
            for j in range(NBUF):
                p = t * NBUF + j
                b = j
                c = (j + 2) % NBUF
                pltpu.make_async_copy(h_sh.at[srcv[b]], rows[b],
                                      sg[b]).wait()
                unpack(p, dstv[b], 14)
                pltpu.async_copy(rows[b], acc.at[dstv[b]], ss[b], add=True)

                @pl.when(p >= 2)
                def _():
                    pltpu.make_async_copy(rows[c], acc.at[dstv[c]],
                                          ss[c]).wait()

                @pl.when(p + 2 < NCHUNK)
                def _():
                    unpack(p + 2, srcv[c], 0)
                    pltpu.async_copy(h_sh.at[srcv[c]], rows[c], sg[c])
            return carry

        lax.fori_loop(0, NCHUNK // NBUF, body, 0)
        for p in (NCHUNK - 2, NCHUNK - 1):
            b = p % NBUF
            pltpu.make_async_copy(rows[b], acc.at[dstv[b]], ss[b]).wait()

    plsc.subcore_barrier()
    pltpu.sync_copy(
        acc.at[pl.ds(sid * RPT, RPT)],
        out_hbm.at[pl.ds(sid * RPT, RPT), pl.ds(cid * DH, DH)])


def _dis(degp_ref):
    return lax.rsqrt(degp_ref[0, :] + degp_ref[1, :] + 1.0)


def _tc_first(x_ref, w_ref, degp_ref, o_ref):
    h = jnp.dot(x_ref[...], w_ref[...], preferred_element_type=jnp.float32)
    o_ref[...] = (h * _dis(degp_ref)[:, None]).astype(jnp.bfloat16)


def _tc_mid(p_ref, h_ref, degp_ref, w_ref, b_ref, o_ref):
    dis = _dis(degp_ref)
    s = p_ref[...].astype(jnp.float32) + h_ref[...].astype(jnp.float32)
    a = jnp.maximum(s * dis[:, None] + b_ref[...], 0.0)
    o_ref[...] = (jnp.dot(a, w_ref[...], preferred_element_type=jnp.float32)
                  * dis[:, None]).astype(jnp.bfloat16)


def _tc_last(p_ref, h_ref, degp_ref, b_ref, o_ref):
    s = p_ref[...].astype(jnp.float32) + h_ref[...].astype(jnp.float32)
    o_ref[...] = s * _dis(degp_ref)[:, None] + b_ref[...]


_SPEC_ROWS = pl.BlockSpec((BLK, D), lambda i: (i, 0))
_SPEC_W = pl.BlockSpec((D, D), lambda i: (0, 0))
_SPEC_DEG = pl.BlockSpec((8, BLK), lambda i: (0, i))
_SPEC_B = pl.BlockSpec((1, D), lambda i: (0, 0))
_OUT_ROWS = jax.ShapeDtypeStruct((N_PAD, D), jnp.float32)
_OUT_BF = jax.ShapeDtypeStruct((N_PAD, D), jnp.bfloat16)


@jax.jit
def kernel(x, edge_index, W1, b1, W2, b2):
    ei = edge_index.astype(jnp.int32)
    x_pad = jnp.pad(x, ((0, N_PAD - N_NODES), (0, 0)))
    z1 = jnp.zeros((N_PAD,), jnp.float32)
    zrows = jnp.zeros((RPT, DH), jnp.bfloat16)
    b1r = b1.reshape(1, D)
    b2r = b2.reshape(1, D)

    degp, packed3 = _sc_prep(ei, z1)

    h1 = pl.pallas_call(
        _tc_first,
        grid=(GRID,),
        in_specs=[_SPEC_ROWS, _SPEC_W, _SPEC_DEG],
        out_specs=_SPEC_ROWS,
        out_shape=_OUT_BF,
    )(x_pad, W1, degp)

    p1 = _sc_scatter(packed3, h1, zrows)

    h2 = pl.pallas_call(
        _tc_mid,
        grid=(GRID,),
        in_specs=[_SPEC_ROWS, _SPEC_ROWS, _SPEC_DEG, _SPEC_W, _SPEC_B],
        out_specs=_SPEC_ROWS,
        out_shape=_OUT_BF,
    )(p1, h1, degp, W2, b1r)

    p2 = _sc_scatter(packed3, h2, zrows)

    out = pl.pallas_call(
        _tc_last,
        grid=(GRID,),
        in_specs=[_SPEC_ROWS, _SPEC_ROWS, _SPEC_DEG, _SPEC_B],
        out_specs=_SPEC_ROWS,
        out_shape=jax.ShapeDtypeStruct((N_NODES, D), jnp.float32),
    )(p2, h2, degp, b2r)

    return out
